# baseline (device time: 219858 ns/iter reference)
import jax
import jax.numpy as jnp
from jax import lax
from jax.experimental import pallas as pl
from jax.experimental.pallas import tpu as pltpu

S_FULL = 2048
S_HALF = 1024
K_DIM = 4096
N_TOTAL = 8192
N_HALF = 4096
BN = 256
K = N_HALF // BN
D_Y = 4
D_X = 6


def _body(o_ref, wo_ref, out_ref,
          send_y, recv_y, send_x, recv_x, near_buf,
          sy_sems, ry_sems, sx_sems, rx_sems):
    k = pl.program_id(0)
    my_x = lax.axis_index("x")
    my_y = lax.axis_index("y")
    y_peer = (my_x, 1 - my_y)
    x_peer = (1 - my_x, my_y)

    def y_rdma(slot):
        return pltpu.make_async_remote_copy(
            src_ref=send_y.at[slot], dst_ref=recv_y.at[slot],
            send_sem=sy_sems.at[slot], recv_sem=ry_sems.at[slot],
            device_id=y_peer, device_id_type=pl.DeviceIdType.MESH,
        )

    def x_rdma(slot):
        return pltpu.make_async_remote_copy(
            src_ref=send_x.at[slot], dst_ref=recv_x.at[slot],
            send_sem=sx_sems.at[slot], recv_sem=rx_sems.at[slot],
            device_id=x_peer, device_id_type=pl.DeviceIdType.MESH,
        )

    @pl.when(k == 0)
    def _():
        barrier_sem = pltpu.get_barrier_semaphore()
        for peer in (y_peer, x_peer):
            pl.semaphore_signal(
                barrier_sem, inc=1, device_id=peer,
                device_id_type=pl.DeviceIdType.MESH,
            )
        pl.semaphore_wait(barrier_sem, 2)

    @pl.when(k < K)
    def _():
        @pl.when(k >= D_Y)
        def _():
            y_rdma(lax.rem(k, D_Y)).wait_send()

        wo16 = wo_ref[...].astype(jnp.bfloat16)
        far = jnp.dot(
            o_ref[pl.ds((1 - my_y) * S_HALF, S_HALF), :],
            wo16,
            preferred_element_type=jnp.float32,
        )
        slot = lax.rem(k, D_Y)
        send_y[slot] = far.astype(jnp.bfloat16)
        y_rdma(slot).start()
        near = jnp.dot(
            o_ref[pl.ds(my_y * S_HALF, S_HALF), :],
            wo16,
            preferred_element_type=jnp.float32,
        )
        near_buf[lax.rem(k, 2)] = near

    @pl.when(jnp.logical_and(k >= 1, k <= K))
    def _():
        c1 = k - 1
        slot = lax.rem(c1, D_Y)
        y_rdma(slot).wait_recv()
        q = near_buf[lax.rem(c1, 2)] + recv_y[slot].astype(jnp.float32)
        q16 = q.astype(jnp.bfloat16)
        out_ref[:, pl.ds(my_x * N_HALF + c1 * BN, BN)] = q16

        xslot = lax.rem(c1, D_X)

        @pl.when(c1 >= D_X)
        def _():
            x_rdma(xslot).wait_send()

        send_x[xslot] = q16
        x_rdma(xslot).start()

    @pl.when(k >= 2)
    def _():
        c2 = k - 2
        xslot = lax.rem(c2, D_X)
        x_rdma(xslot).wait_recv()
        out_ref[:, pl.ds((1 - my_x) * N_HALF + c2 * BN, BN)] = recv_x[xslot]

    @pl.when(k == K + 1)
    def _():
        for s in range(D_Y):
            y_rdma(s).wait_send()
        for s in range(D_X):
            x_rdma(s).wait_send()


def kernel(O, Wo):
    O2 = O.reshape(S_FULL, K_DIM).astype(jnp.bfloat16)

    out = pl.pallas_call(
        _body,
        grid=(K + 2,),
        in_specs=[
            pl.BlockSpec((S_FULL, K_DIM), lambda k: (0, 0)),
            pl.BlockSpec(
                (K_DIM, BN),
                lambda k: (
                    0, lax.axis_index("x") * K + jnp.minimum(k, K - 1)
                ),
            ),
        ],
        out_specs=pl.BlockSpec((S_HALF, N_TOTAL), lambda k: (0, 0)),
        out_shape=jax.ShapeDtypeStruct((S_HALF, N_TOTAL), jnp.bfloat16),
        scratch_shapes=[
            pltpu.VMEM((D_Y, S_HALF, BN), jnp.bfloat16),
            pltpu.VMEM((D_Y, S_HALF, BN), jnp.bfloat16),
            pltpu.VMEM((D_X, S_HALF, BN), jnp.bfloat16),
            pltpu.VMEM((D_X, S_HALF, BN), jnp.bfloat16),
            pltpu.VMEM((2, S_HALF, BN), jnp.float32),
            pltpu.SemaphoreType.DMA((D_Y,)),
            pltpu.SemaphoreType.DMA((D_Y,)),
            pltpu.SemaphoreType.DMA((D_X,)),
            pltpu.SemaphoreType.DMA((D_X,)),
        ],
        compiler_params=pltpu.CompilerParams(
            collective_id=0,
            vmem_limit_bytes=64 * 1024 * 1024,
        ),
    )(O2, Wo)
    return out.astype(jnp.float32).reshape(1, S_HALF, N_TOTAL)


# device time: 175311 ns/iter; 1.2541x vs baseline; 1.2541x over previous
import jax
import jax.numpy as jnp
from jax import lax
from jax.experimental import pallas as pl
from jax.experimental.pallas import tpu as pltpu

S_FULL = 2048
S_HALF = 1024
K_DIM = 4096
N_TOTAL = 8192
N_HALF = 4096
BN = 256
K = N_HALF // BN
D_Y = 4
D_X = 6


def _body(o_ref, wo_ref, out_ref,
          send_y, recv_y, send_x, recv_x, near_buf,
          sy_sems, ry_sems, sx_sems, rx_sems):
    k = pl.program_id(0)
    my_x = lax.axis_index("x")
    my_y = lax.axis_index("y")
    y_peer = (my_x, 1 - my_y)
    x_peer = (1 - my_x, my_y)

    def y_rdma(slot):
        return pltpu.make_async_remote_copy(
            src_ref=send_y.at[slot], dst_ref=recv_y.at[slot],
            send_sem=sy_sems.at[slot], recv_sem=ry_sems.at[slot],
            device_id=y_peer, device_id_type=pl.DeviceIdType.MESH,
        )

    def x_rdma(slot):
        return pltpu.make_async_remote_copy(
            src_ref=send_x.at[slot], dst_ref=recv_x.at[slot],
            send_sem=sx_sems.at[slot], recv_sem=rx_sems.at[slot],
            device_id=x_peer, device_id_type=pl.DeviceIdType.MESH,
        )

    @pl.when(k == 0)
    def _():
        barrier_sem = pltpu.get_barrier_semaphore()
        for peer in (y_peer, x_peer):
            pl.semaphore_signal(
                barrier_sem, inc=1, device_id=peer,
                device_id_type=pl.DeviceIdType.MESH,
            )
        pl.semaphore_wait(barrier_sem, 2)

    @pl.when(k < K)
    def _():
        @pl.when(k >= D_Y)
        def _():
            y_rdma(lax.rem(k, D_Y)).wait_send()

        wo16 = wo_ref[...].astype(jnp.bfloat16)
        far = jnp.dot(
            o_ref[pl.ds((1 - my_y) * S_HALF, S_HALF), :],
            wo16,
            preferred_element_type=jnp.float32,
        )
        near = jnp.dot(
            o_ref[pl.ds(my_y * S_HALF, S_HALF), :],
            wo16,
            preferred_element_type=jnp.float32,
        )
        slot = lax.rem(k, D_Y)
        send_y[slot] = far.astype(jnp.bfloat16)
        y_rdma(slot).start()
        near_buf[lax.rem(k, 2)] = near

    @pl.when(jnp.logical_and(k >= 1, k <= K))
    def _():
        c1 = k - 1
        slot = lax.rem(c1, D_Y)
        y_rdma(slot).wait_recv()
        q = near_buf[lax.rem(c1, 2)] + recv_y[slot].astype(jnp.float32)
        q16 = q.astype(jnp.bfloat16)
        out_ref[:, pl.ds(my_x * N_HALF + c1 * BN, BN)] = q16

        xslot = lax.rem(c1, D_X)

        @pl.when(c1 >= D_X)
        def _():
            x_rdma(xslot).wait_send()

        send_x[xslot] = q16
        x_rdma(xslot).start()

    @pl.when(k >= 2)
    def _():
        c2 = k - 2
        xslot = lax.rem(c2, D_X)
        x_rdma(xslot).wait_recv()
        out_ref[:, pl.ds((1 - my_x) * N_HALF + c2 * BN, BN)] = recv_x[xslot]

    @pl.when(k == K + 1)
    def _():
        for s in range(D_Y):
            y_rdma(s).wait_send()
        for s in range(D_X):
            x_rdma(s).wait_send()


def kernel(O, Wo):
    O2 = O.reshape(S_FULL, K_DIM).astype(jnp.bfloat16)

    out = pl.pallas_call(
        _body,
        grid=(K + 2,),
        in_specs=[
            pl.BlockSpec((S_FULL, K_DIM), lambda k: (0, 0)),
            pl.BlockSpec(
                (K_DIM, BN),
                lambda k: (
                    0, lax.axis_index("x") * K + jnp.minimum(k, K - 1)
                ),
            ),
        ],
        out_specs=pl.BlockSpec((S_HALF, N_TOTAL), lambda k: (0, 0)),
        out_shape=jax.ShapeDtypeStruct((S_HALF, N_TOTAL), jnp.bfloat16),
        scratch_shapes=[
            pltpu.VMEM((D_Y, S_HALF, BN), jnp.bfloat16),
            pltpu.VMEM((D_Y, S_HALF, BN), jnp.bfloat16),
            pltpu.VMEM((D_X, S_HALF, BN), jnp.bfloat16),
            pltpu.VMEM((D_X, S_HALF, BN), jnp.bfloat16),
            pltpu.VMEM((2, S_HALF, BN), jnp.float32),
            pltpu.SemaphoreType.DMA((D_Y,)),
            pltpu.SemaphoreType.DMA((D_Y,)),
            pltpu.SemaphoreType.DMA((D_X,)),
            pltpu.SemaphoreType.DMA((D_X,)),
        ],
        compiler_params=pltpu.CompilerParams(
            collective_id=0,
            vmem_limit_bytes=64 * 1024 * 1024,
        ),
    )(O2, Wo)
    return out.astype(jnp.float32).reshape(1, S_HALF, N_TOTAL)


# device time: 165240 ns/iter; 1.3305x vs baseline; 1.0609x over previous
import jax
import jax.numpy as jnp
from jax import lax
from jax.experimental import pallas as pl
from jax.experimental.pallas import tpu as pltpu

S_FULL = 2048
S_HALF = 1024
K_DIM = 4096
N_TOTAL = 8192
N_HALF = 4096
BN = 256
K = N_HALF // BN
LAG_Y = 2
LAG_X = 4
D_Y = 6
D_X = 8


def _body(o_ref, wo_ref, out_ref,
          send_y, recv_y, send_x, recv_x,
          sy_sems, ry_sems, sx_sems, rx_sems):
    k = pl.program_id(0)
    my_x = lax.axis_index("x")
    my_y = lax.axis_index("y")
    y_peer = (my_x, 1 - my_y)
    x_peer = (1 - my_x, my_y)

    def y_rdma(slot):
        return pltpu.make_async_remote_copy(
            src_ref=send_y.at[slot], dst_ref=recv_y.at[slot],
            send_sem=sy_sems.at[slot], recv_sem=ry_sems.at[slot],
            device_id=y_peer, device_id_type=pl.DeviceIdType.MESH,
        )

    def x_rdma(slot):
        return pltpu.make_async_remote_copy(
            src_ref=send_x.at[slot], dst_ref=recv_x.at[slot],
            send_sem=sx_sems.at[slot], recv_sem=rx_sems.at[slot],
            device_id=x_peer, device_id_type=pl.DeviceIdType.MESH,
        )

    @pl.when(k == 0)
    def _():
        barrier_sem = pltpu.get_barrier_semaphore()
        for peer in (y_peer, x_peer):
            pl.semaphore_signal(
                barrier_sem, inc=1, device_id=peer,
                device_id_type=pl.DeviceIdType.MESH,
            )
        pl.semaphore_wait(barrier_sem, 2)

    @pl.when(k < K)
    def _():
        @pl.when(k >= D_Y)
        def _():
            y_rdma(lax.rem(k, D_Y)).wait_send()

        wo16 = wo_ref[...].astype(jnp.bfloat16)
        far = jnp.dot(
            o_ref[pl.ds((1 - my_y) * S_HALF, S_HALF), :],
            wo16,
            preferred_element_type=jnp.float32,
        )
        near = jnp.dot(
            o_ref[pl.ds(my_y * S_HALF, S_HALF), :],
            wo16,
            preferred_element_type=jnp.float32,
        )
        slot = lax.rem(k, D_Y)
        send_y[slot] = far.astype(jnp.bfloat16)
        y_rdma(slot).start()
        out_ref[:, pl.ds(my_x * N_HALF + k * BN, BN)] = near.astype(
            jnp.bfloat16
        )

    @pl.when(jnp.logical_and(k >= LAG_Y, k <= K + LAG_Y - 1))
    def _():
        c1 = k - LAG_Y
        slot = lax.rem(c1, D_Y)
        y_rdma(slot).wait_recv()
        col = pl.ds(my_x * N_HALF + c1 * BN, BN)
        q = (out_ref[:, col].astype(jnp.float32)
             + recv_y[slot].astype(jnp.float32))
        q16 = q.astype(jnp.bfloat16)
        out_ref[:, col] = q16

        xslot = lax.rem(c1, D_X)

        @pl.when(c1 >= D_X)
        def _():
            x_rdma(xslot).wait_send()

        send_x[xslot] = q16
        x_rdma(xslot).start()

    @pl.when(k >= LAG_X)
    def _():
        c2 = k - LAG_X
        xslot = lax.rem(c2, D_X)
        x_rdma(xslot).wait_recv()
        out_ref[:, pl.ds((1 - my_x) * N_HALF + c2 * BN, BN)] = recv_x[xslot]

    @pl.when(k == K + LAG_X - 1)
    def _():
        for s in range(D_Y):
            y_rdma(s).wait_send()
        for s in range(D_X):
            x_rdma(s).wait_send()


def kernel(O, Wo):
    O2 = O.reshape(S_FULL, K_DIM).astype(jnp.bfloat16)

    out = pl.pallas_call(
        _body,
        grid=(K + LAG_X,),
        in_specs=[
            pl.BlockSpec((S_FULL, K_DIM), lambda k: (0, 0)),
            pl.BlockSpec(
                (K_DIM, BN),
                lambda k: (
                    0, lax.axis_index("x") * K + jnp.minimum(k, K - 1)
                ),
            ),
        ],
        out_specs=pl.BlockSpec((S_HALF, N_TOTAL), lambda k: (0, 0)),
        out_shape=jax.ShapeDtypeStruct((S_HALF, N_TOTAL), jnp.bfloat16),
        scratch_shapes=[
            pltpu.VMEM((D_Y, S_HALF, BN), jnp.bfloat16),
            pltpu.VMEM((D_Y, S_HALF, BN), jnp.bfloat16),
            pltpu.VMEM((D_X, S_HALF, BN), jnp.bfloat16),
            pltpu.VMEM((D_X, S_HALF, BN), jnp.bfloat16),
            pltpu.SemaphoreType.DMA((D_Y,)),
            pltpu.SemaphoreType.DMA((D_Y,)),
            pltpu.SemaphoreType.DMA((D_X,)),
            pltpu.SemaphoreType.DMA((D_X,)),
        ],
        compiler_params=pltpu.CompilerParams(
            collective_id=0,
            vmem_limit_bytes=64 * 1024 * 1024,
        ),
    )(O2, Wo)
    return out.astype(jnp.float32).reshape(1, S_HALF, N_TOTAL)
